# baseline (speedup 1.0000x reference)
import functools

import jax
import jax.numpy as jnp
import numpy as np
from jax import lax
from jax.experimental import pallas as pl
from jax.experimental.pallas import tpu as pltpu
from jax.experimental.pallas import tpu_sc as plsc

_N = 10000
_NP = 10240
_K = 16
_C = 128
_RB = 256
_RB2 = 256
_BIGI = np.int32(2**30)


_NCH = _NP // 128
_CW = _K * _NCH


def _t_step(c_ref, f_ref, w1p_ref, w1f_ref, t_ref):
    xr = c_ref[...]
    w1p = w1p_ref[...]
    tp = (xr[:, 0:1] * w1p[0:1, :] + xr[:, 1:2] * w1p[1:2, :]
          + xr[:, 2:3] * w1p[2:3, :])
    t_ref[...] = tp + jnp.dot(f_ref[...], w1f_ref[...],
                              preferred_element_type=jnp.float32)


def _t_call(coords_p, features_p, w1p, w1f):
    return pl.pallas_call(
        _t_step,
        grid=(_NP // _RB,),
        in_specs=[
            pl.BlockSpec((_RB, 3), lambda i: (i, 0)),
            pl.BlockSpec((_RB, _C), lambda i: (i, 0)),
            pl.BlockSpec((3, _C), lambda i: (0, 0)),
            pl.BlockSpec((_C, _C), lambda i: (0, 0)),
        ],
        out_specs=pl.BlockSpec((_RB, _C), lambda i: (i, 0)),
        out_shape=jax.ShapeDtypeStruct((_NP, _C), jnp.float32),
    )(coords_p, features_p, w1p, w1f)


def _knn_step(c_ref, ct_ref, idx_ref):
    xr = c_ref[...]
    ct = ct_ref[...]
    x0 = xr[:, 0:1]
    x1 = xr[:, 1:2]
    x2 = xr[:, 2:3]
    dot = jnp.dot(xr, ct, preferred_element_type=jnp.float32)
    sq_r = x0 * x0 + x1 * x1 + x2 * x2
    sq_c = (ct[0:1, :] * ct[0:1, :] + ct[1:2, :] * ct[1:2, :]
            + ct[2:3, :] * ct[2:3, :])
    d2 = (sq_r + sq_c) - 2.0 * dot

    accs = [d2[:, v * 128:(v + 1) * 128] for v in range(8)]
    for v in range(8, _NCH):
        accs[v % 8] = jnp.minimum(accs[v % 8], d2[:, v * 128:(v + 1) * 128])
    a0 = jnp.minimum(jnp.minimum(accs[0], accs[1]),
                     jnp.minimum(accs[2], accs[3]))
    a1 = jnp.minimum(jnp.minimum(accs[4], accs[5]),
                     jnp.minimum(accs[6], accs[7]))
    cmin = jnp.minimum(a0, a1)

    def tmin(x):
        s = x.shape[0]
        while s > 8 and s % 2 == 0:
            s //= 2
            x = jnp.minimum(x[:s], x[s:])
        return jnp.min(x, axis=0, keepdims=True)

    sub = lax.broadcasted_iota(jnp.int32, (128, _RB), 0)
    sub16 = lax.broadcasted_iota(jnp.int32, (_K, _RB), 0)

    def top16_classes(cmt):
        lst = jnp.zeros((_K, _RB), jnp.int32)
        for s in range(_K):
            m = tmin(cmt)
            hit = cmt == m
            lsel = tmin(jnp.where(hit, sub, 128))
            cmt = jnp.where(sub == lsel, jnp.inf, cmt)
            lst = jnp.where(sub16 == s, lsel, lst)
        return lst

    ls = top16_classes(cmin.T).T

    p = lax.broadcasted_iota(jnp.int32, (_RB, _CW), 1)
    s_part = p & (_K - 1)
    ls_g = jnp.take_along_axis(ls, s_part, axis=1)
    cols = ((p >> 4) << 7) + ls_g

    cand = jnp.concatenate(
        [jnp.take_along_axis(d2[:, v * 128:(v + 1) * 128], ls, axis=1)
         for v in range(_NCH)], axis=1)

    nt2 = _CW // 128
    cm2 = cand[:, 0:128]
    for u in range(1, nt2):
        cm2 = jnp.minimum(cm2, cand[:, u * 128:(u + 1) * 128])

    ls2 = top16_classes(cm2.T).T

    vp = [jnp.take_along_axis(cand[:, u * 128:(u + 1) * 128], ls2, axis=1)
          for u in range(nt2)]
    cp = [jnp.take_along_axis(cols[:, u * 128:(u + 1) * 128], ls2, axis=1)
          for u in range(nt2)]
    cand2 = jnp.concatenate(vp, axis=1)
    cols2 = jnp.concatenate(cp, axis=1)

    vt = cand2.T
    gt = cols2.T
    idxt = jnp.zeros((_K, _RB), jnp.int32)
    for t in range(_K):
        m = tmin(vt)
        hit = vt == m
        sel = tmin(jnp.where(hit, gt, _BIGI))
        vt = jnp.where(gt == sel, jnp.inf, vt)
        idxt = jnp.where(sub16 == t, sel, idxt)
    idx_ref[...] = idxt.T


def _mlp_step(g_ref, c_ref, f_ref, w1p_ref, b1_ref, w2_ref, b2_ref,
              ws_ref, bs_ref, o_ref):
    xr = c_ref[...]
    w1p = w1p_ref[...]
    q = (xr[:, 0:1] * w1p[0:1, :] + xr[:, 1:2] * w1p[1:2, :]
         + xr[:, 2:3] * w1p[2:3, :])
    g = g_ref[...].reshape(_RB2, _K, _C)
    pre = g - q[:, None, :] + b1_ref[...][None]
    act = 0.5 * pre * (1.0 + lax.erf(pre * np.float32(np.sqrt(0.5))))
    s = jnp.sum(act, axis=1) * np.float32(1.0 / _K)
    agg = jnp.dot(s, w2_ref[...], preferred_element_type=jnp.float32) + b2_ref[...]
    o_ref[...] = agg + jnp.dot(f_ref[...], ws_ref[...],
                               preferred_element_type=jnp.float32) + bs_ref[...]


def _knn_call(coords_half, ct):
    nh = coords_half.shape[0]
    return pl.pallas_call(
        _knn_step,
        grid=(nh // _RB,),
        in_specs=[
            pl.BlockSpec((_RB, 3), lambda i: (i, 0)),
            pl.BlockSpec((3, _NP), lambda i: (0, 0)),
        ],
        out_specs=pl.BlockSpec((_RB, _K), lambda i: (i, 0)),
        out_shape=jax.ShapeDtypeStruct((nh, _K), jnp.int32),
        compiler_params=pltpu.CompilerParams(
            dimension_semantics=("arbitrary",)),
    )(coords_half, ct)


def _mlp_call(g, coords_h, features_h, w1p, b1, w2, b2, ws, bs):
    nh = coords_h.shape[0]
    return pl.pallas_call(
        _mlp_step,
        grid=(nh // _RB2,),
        in_specs=[
            pl.BlockSpec((_RB2 * _K, _C), lambda i: (i, 0)),
            pl.BlockSpec((_RB2, 3), lambda i: (i, 0)),
            pl.BlockSpec((_RB2, _C), lambda i: (i, 0)),
            pl.BlockSpec((3, _C), lambda i: (0, 0)),
            pl.BlockSpec((1, _C), lambda i: (0, 0)),
            pl.BlockSpec((_C, _C), lambda i: (0, 0)),
            pl.BlockSpec((1, _C), lambda i: (0, 0)),
            pl.BlockSpec((_C, _C), lambda i: (0, 0)),
            pl.BlockSpec((1, _C), lambda i: (0, 0)),
        ],
        out_specs=pl.BlockSpec((_RB2, _C), lambda i: (i, 0)),
        out_shape=jax.ShapeDtypeStruct((nh, _C), jnp.float32),
    )(g, coords_h, features_h, w1p, b1, w2, b2, ws, bs)


def _gather_call(table, idx_flat):
    info = plsc.get_sparse_core_info()
    nc, ns = info.num_cores, info.num_subcores
    nw = nc * ns
    b = idx_flat.shape[0]
    per_w = b // nw
    ch = 128
    nbuf = 5
    n_grp = per_w // (ch * nbuf)
    mesh = plsc.VectorSubcoreMesh(core_axis_name="c", subcore_axis_name="s")

    @functools.partial(
        pl.kernel,
        out_type=jax.ShapeDtypeStruct((b, _C), jnp.float32),
        mesh=mesh,
        scratch_types=(
            [pltpu.VMEM((ch,), jnp.int32)] * nbuf
            + [pltpu.VMEM((ch, _C), jnp.float32)] * nbuf
            + [pltpu.SemaphoreType.DMA]
        ),
    )
    def gk(tbl_hbm, idx_hbm, out_hbm,
           i0, i1, i2, i3, i4, r0, r1, r2, r3, r4, sem):
        wid = lax.axis_index("s") * nc + lax.axis_index("c")
        base = wid * per_w
        idxs = (i0, i1, i2, i3, i4)
        rows = (r0, r1, r2, r3, r4)

        def body(g, carry):
            goff = base + g * (ch * nbuf)
            for bb in range(nbuf):
                pltpu.sync_copy(idx_hbm.at[pl.ds(goff + bb * ch, ch)],
                                idxs[bb])
            cps = [pltpu.async_copy(tbl_hbm.at[idxs[bb]], rows[bb], sem)
                   for bb in range(nbuf)]
            for bb in range(nbuf):
                cps[bb].wait()
                pltpu.sync_copy(rows[bb],
                                out_hbm.at[pl.ds(goff + bb * ch, ch)])
            return carry

        lax.fori_loop(0, n_grp, body, 0)

    return gk(table, idx_flat)


def kernel(coords, features, W1, b1, W2, b2, Ws, bs, offset):
    del offset
    coords_p = jnp.pad(coords, ((0, _NP - _N), (0, 0)),
                       constant_values=np.float32(1e4))
    features_p = jnp.pad(features, ((0, _NP - _N), (0, 0)))
    ct = coords_p.T
    w1p = W1[:3]
    w1f = W1[3:]
    t = _t_call(coords_p, features_p, w1p, w1f)
    nq = 4
    h = _NP // nq
    idxs = []
    gs = []
    for q in range(nq):
        idxq = _knn_call(coords_p[q * h:(q + 1) * h], ct)
        gs.append(_gather_call(t, idxq.reshape(-1)))
    outs = [
        _mlp_call(gs[q], coords_p[q * h:(q + 1) * h],
                  features_p[q * h:(q + 1) * h], w1p, b1[None, :], W2,
                  b2[None, :], Ws, bs[None, :])
        for q in range(nq)
    ]
    return jnp.concatenate(outs, axis=0)[:_N]


# speedup vs baseline: 1.0068x; 1.0068x over previous
import functools

import jax
import jax.numpy as jnp
import numpy as np
from jax import lax
from jax.experimental import pallas as pl
from jax.experimental.pallas import tpu as pltpu
from jax.experimental.pallas import tpu_sc as plsc

_N = 10000
_NP = 10240
_K = 16
_C = 128
_RB = 256
_RB2 = 256
_BIGI = np.int32(2**30)


_NCH = _NP // 128
_CW = _K * _NCH


def _t_step(c_ref, f_ref, w1p_ref, w1f_ref, t_ref):
    xr = c_ref[...]
    w1p = w1p_ref[...]
    tp = (xr[:, 0:1] * w1p[0:1, :] + xr[:, 1:2] * w1p[1:2, :]
          + xr[:, 2:3] * w1p[2:3, :])
    t_ref[...] = tp + jnp.dot(f_ref[...], w1f_ref[...],
                              preferred_element_type=jnp.float32)


def _t_call(coords_p, features_p, w1p, w1f):
    return pl.pallas_call(
        _t_step,
        grid=(_NP // _RB,),
        in_specs=[
            pl.BlockSpec((_RB, 3), lambda i: (i, 0)),
            pl.BlockSpec((_RB, _C), lambda i: (i, 0)),
            pl.BlockSpec((3, _C), lambda i: (0, 0)),
            pl.BlockSpec((_C, _C), lambda i: (0, 0)),
        ],
        out_specs=pl.BlockSpec((_RB, _C), lambda i: (i, 0)),
        out_shape=jax.ShapeDtypeStruct((_NP, _C), jnp.float32),
    )(coords_p, features_p, w1p, w1f)


def _knn_step(c_ref, ct_ref, idx_ref):
    xr = c_ref[...]
    ct = ct_ref[...]
    x0 = xr[:, 0:1]
    x1 = xr[:, 1:2]
    x2 = xr[:, 2:3]
    dot = jnp.dot(xr, ct, preferred_element_type=jnp.float32)
    sq_r = x0 * x0 + x1 * x1 + x2 * x2
    sq_c = (ct[0:1, :] * ct[0:1, :] + ct[1:2, :] * ct[1:2, :]
            + ct[2:3, :] * ct[2:3, :])
    d2 = (sq_r + sq_c) - 2.0 * dot

    accs = [d2[:, v * 128:(v + 1) * 128] for v in range(8)]
    for v in range(8, _NCH):
        accs[v % 8] = jnp.minimum(accs[v % 8], d2[:, v * 128:(v + 1) * 128])
    a0 = jnp.minimum(jnp.minimum(accs[0], accs[1]),
                     jnp.minimum(accs[2], accs[3]))
    a1 = jnp.minimum(jnp.minimum(accs[4], accs[5]),
                     jnp.minimum(accs[6], accs[7]))
    cmin = jnp.minimum(a0, a1)

    def tmin(x):
        s = x.shape[0]
        while s > 8 and s % 2 == 0:
            s //= 2
            x = jnp.minimum(x[:s], x[s:])
        return jnp.min(x, axis=0, keepdims=True)

    sub = lax.broadcasted_iota(jnp.int32, (128, _RB), 0)
    sub16 = lax.broadcasted_iota(jnp.int32, (_K, _RB), 0)

    def top16_classes(cmt):
        lst = jnp.zeros((_K, _RB), jnp.int32)
        for s in range(_K):
            m = tmin(cmt)
            hit = cmt == m
            lsel = tmin(jnp.where(hit, sub, 128))
            cmt = jnp.where(sub == lsel, jnp.inf, cmt)
            lst = jnp.where(sub16 == s, lsel, lst)
        return lst

    ls = top16_classes(cmin.T).T

    p = lax.broadcasted_iota(jnp.int32, (_RB, _CW), 1)
    s_part = p & (_K - 1)
    ls_g = jnp.take_along_axis(ls, s_part, axis=1)
    cols = ((p >> 4) << 7) + ls_g

    cand = jnp.concatenate(
        [jnp.take_along_axis(d2[:, v * 128:(v + 1) * 128], ls, axis=1)
         for v in range(_NCH)], axis=1)

    nt2 = _CW // 128
    cm2 = cand[:, 0:128]
    for u in range(1, nt2):
        cm2 = jnp.minimum(cm2, cand[:, u * 128:(u + 1) * 128])

    ls2 = top16_classes(cm2.T).T

    vp = [jnp.take_along_axis(cand[:, u * 128:(u + 1) * 128], ls2, axis=1)
          for u in range(nt2)]
    cp = [jnp.take_along_axis(cols[:, u * 128:(u + 1) * 128], ls2, axis=1)
          for u in range(nt2)]
    cand2 = jnp.concatenate(vp, axis=1)
    cols2 = jnp.concatenate(cp, axis=1)

    vt = cand2.T
    gt = cols2.T
    idxt = jnp.zeros((_K, _RB), jnp.int32)
    for t in range(_K):
        m = tmin(vt)
        hit = vt == m
        sel = tmin(jnp.where(hit, gt, _BIGI))
        vt = jnp.where(gt == sel, jnp.inf, vt)
        idxt = jnp.where(sub16 == t, sel, idxt)
    idx_ref[...] = idxt.T


def _mlp_step(g_ref, c_ref, f_ref, w1p_ref, b1_ref, w2_ref, b2_ref,
              ws_ref, bs_ref, o_ref):
    xr = c_ref[...]
    w1p = w1p_ref[...]
    q = (xr[:, 0:1] * w1p[0:1, :] + xr[:, 1:2] * w1p[1:2, :]
         + xr[:, 2:3] * w1p[2:3, :])
    g = g_ref[...].reshape(_RB2, _K, _C)
    pre = g - q[:, None, :] + b1_ref[...][None]
    act = 0.5 * pre * (1.0 + lax.erf(pre * np.float32(np.sqrt(0.5))))
    s = jnp.sum(act, axis=1) * np.float32(1.0 / _K)
    agg = jnp.dot(s, w2_ref[...], preferred_element_type=jnp.float32) + b2_ref[...]
    o_ref[...] = agg + jnp.dot(f_ref[...], ws_ref[...],
                               preferred_element_type=jnp.float32) + bs_ref[...]


def _knn_call(coords_half, ct):
    nh = coords_half.shape[0]
    return pl.pallas_call(
        _knn_step,
        grid=(nh // _RB,),
        in_specs=[
            pl.BlockSpec((_RB, 3), lambda i: (i, 0)),
            pl.BlockSpec((3, _NP), lambda i: (0, 0)),
        ],
        out_specs=pl.BlockSpec((_RB, _K), lambda i: (i, 0)),
        out_shape=jax.ShapeDtypeStruct((nh, _K), jnp.int32),
        compiler_params=pltpu.CompilerParams(
            dimension_semantics=("arbitrary",)),
    )(coords_half, ct)


def _mlp_call(g, coords_h, features_h, w1p, b1, w2, b2, ws, bs):
    nh = coords_h.shape[0]
    return pl.pallas_call(
        _mlp_step,
        grid=(nh // _RB2,),
        in_specs=[
            pl.BlockSpec((_RB2 * _K, _C), lambda i: (i, 0)),
            pl.BlockSpec((_RB2, 3), lambda i: (i, 0)),
            pl.BlockSpec((_RB2, _C), lambda i: (i, 0)),
            pl.BlockSpec((3, _C), lambda i: (0, 0)),
            pl.BlockSpec((1, _C), lambda i: (0, 0)),
            pl.BlockSpec((_C, _C), lambda i: (0, 0)),
            pl.BlockSpec((1, _C), lambda i: (0, 0)),
            pl.BlockSpec((_C, _C), lambda i: (0, 0)),
            pl.BlockSpec((1, _C), lambda i: (0, 0)),
        ],
        out_specs=pl.BlockSpec((_RB2, _C), lambda i: (i, 0)),
        out_shape=jax.ShapeDtypeStruct((nh, _C), jnp.float32),
    )(g, coords_h, features_h, w1p, b1, w2, b2, ws, bs)


def _gather_call(table, idx_flat):
    info = plsc.get_sparse_core_info()
    nc, ns = info.num_cores, info.num_subcores
    nw = nc * ns
    b = idx_flat.shape[0]
    per_w = b // nw
    ch = 128
    nbuf = 5
    n_grp = per_w // (ch * nbuf)
    mesh = plsc.VectorSubcoreMesh(core_axis_name="c", subcore_axis_name="s")

    @functools.partial(
        pl.kernel,
        out_type=jax.ShapeDtypeStruct((b, _C), jnp.float32),
        mesh=mesh,
        scratch_types=(
            [pltpu.VMEM((ch,), jnp.int32)] * nbuf
            + [pltpu.VMEM((ch, _C), jnp.float32)] * nbuf
            + [pltpu.SemaphoreType.DMA]
        ),
    )
    def gk(tbl_hbm, idx_hbm, out_hbm,
           i0, i1, i2, i3, i4, r0, r1, r2, r3, r4, sem):
        wid = lax.axis_index("s") * nc + lax.axis_index("c")
        base = wid * per_w
        idxs = (i0, i1, i2, i3, i4)
        rows = (r0, r1, r2, r3, r4)

        def body(g, carry):
            goff = base + g * (ch * nbuf)
            for bb in range(nbuf):
                pltpu.sync_copy(idx_hbm.at[pl.ds(goff + bb * ch, ch)],
                                idxs[bb])
            cps = [pltpu.async_copy(tbl_hbm.at[idxs[bb]], rows[bb], sem)
                   for bb in range(nbuf)]
            for bb in range(nbuf):
                cps[bb].wait()
                pltpu.sync_copy(rows[bb],
                                out_hbm.at[pl.ds(goff + bb * ch, ch)])
            return carry

        lax.fori_loop(0, n_grp, body, 0)

    return gk(table, idx_flat)


def kernel(coords, features, W1, b1, W2, b2, Ws, bs, offset):
    del offset
    coords_p = jnp.pad(coords, ((0, _NP - _N), (0, 0)),
                       constant_values=np.float32(1e4))
    features_p = jnp.pad(features, ((0, _NP - _N), (0, 0)))
    ct = coords_p.T
    w1p = W1[:3]
    w1f = W1[3:]
    t = _t_call(coords_p, features_p, w1p, w1f)
    nq = 2
    h = _NP // nq
    gs = []
    for q in range(nq):
        idxq = _knn_call(coords_p[q * h:(q + 1) * h], ct)
        gs.append(_gather_call(t, idxq.reshape(-1)))
    outs = [
        _mlp_call(gs[q], coords_p[q * h:(q + 1) * h],
                  features_p[q * h:(q + 1) * h], w1p, b1[None, :], W2,
                  b2[None, :], Ws, bs[None, :])
        for q in range(nq)
    ]
    return jnp.concatenate(outs, axis=0)[:_N]
